# Initial kernel scaffold; baseline (speedup 1.0000x reference)
import jax
import jax.numpy as jnp
from jax.experimental import pallas as pl


def kernel(x, router_W, router_b, expert_params):
    raise NotImplementedError("write your pallas kernel here")


# speedup vs baseline: 1.8682x; 1.8682x over previous
import functools

import jax
import jax.numpy as jnp
import numpy as np
from jax import lax
from jax.experimental import pallas as pl
from jax.experimental.pallas import tpu as pltpu

_EPS = 1e-5
_NEG = -1e30


def _ffn_body(etab_ref, x_ref, wr_ref, br_ref, b2c_ref, w1_ref, b1_ref,
              w2_ref, o_ref, coef_scr, *, n_experts):
    h_id = pl.program_id(1)

    @pl.when(h_id == 0)
    def _router():
        x = x_ref[...]
        logits = jnp.dot(x, wr_ref[...].T, preferred_element_type=jnp.float32)
        logits = logits + br_ref[...]
        col = lax.broadcasted_iota(jnp.int32, logits.shape, 1)
        lm = jnp.where(col < n_experts, logits, _NEG)
        m1 = jnp.max(lm, axis=1, keepdims=True)
        i1 = jnp.min(jnp.where(lm == m1, col, 127), axis=1, keepdims=True)
        l2 = jnp.where(col == i1, _NEG, lm)
        m2 = jnp.max(l2, axis=1, keepdims=True)
        i2 = jnp.min(jnp.where(l2 == m2, col, 127), axis=1, keepdims=True)
        e = jnp.exp(m2 - m1)
        wa = 1.0 / (1.0 + e)
        wb = 1.0 - wa
        coef = wa * (col == i1) + wb * (col == i2)
        coef_scr[...] = coef
        o_ref[...] = x + jnp.dot(coef, b2c_ref[...],
                                 preferred_element_type=jnp.float32)

    et = etab_ref[h_id]
    coef = coef_scr[...]
    col = lax.broadcasted_iota(jnp.int32, coef.shape, 1)
    csel = jnp.sum(jnp.where(col == et, coef, 0.0), axis=1)
    h = jnp.dot(x_ref[...], w1_ref[...].T, preferred_element_type=jnp.float32)
    h = jnp.maximum(h + b1_ref[...], 0.0) * csel[:, None]
    o_ref[...] += jnp.dot(h, w2_ref[...], preferred_element_type=jnp.float32)


def kernel(x, router_W, router_b, expert_params):
    n, d = x.shape
    n_experts = len(expert_params)
    inv_c = 1.0 / np.sqrt(1.0 + _EPS)

    bm = min(256, n)
    ht = 512

    w1_parts, b1_parts, w2_parts = [], [], []
    b2_rows = []
    sizes_p = []
    for (W1, b1, g1, be1, W2, b2, g2, be2) in expert_params:
        s = W1.shape[0]
        sp = ((s + ht - 1) // ht) * ht
        sizes_p.append(sp)
        s1 = g1 * inv_c
        w1f = W1 * s1[:, None]
        b1f = b1 * s1 + be1
        s2 = g2 * inv_c
        w2f = (W2 * s2[:, None]).T
        b2f = b2 * s2 + be2
        w1_parts.append(jnp.pad(w1f, ((0, sp - s), (0, 0))))
        b1_parts.append(jnp.pad(b1f, (0, sp - s)))
        w2_parts.append(jnp.pad(w2f, ((0, sp - s), (0, 0))))
        b2_rows.append(b2f)

    w1cat = jnp.concatenate(w1_parts, axis=0)
    b1cat = jnp.concatenate(b1_parts, axis=0)[None, :]
    w2cat = jnp.concatenate(w2_parts, axis=0)
    s_tot = w1cat.shape[0]
    nt = s_tot // ht

    etab = np.repeat(np.arange(n_experts, dtype=np.int32),
                     [sp // ht for sp in sizes_p])

    wr_pad = jnp.pad(router_W, ((0, 128 - n_experts), (0, 0)))
    br_pad = jnp.pad(router_b, (0, 128 - n_experts))[None, :]
    b2c = jnp.pad(jnp.stack(b2_rows, axis=0), ((0, 128 - n_experts), (0, 0)))

    nb = n // bm
    grid_spec = pltpu.PrefetchScalarGridSpec(
        num_scalar_prefetch=1,
        grid=(nb, nt),
        in_specs=[
            pl.BlockSpec((bm, d), lambda b, h, tab: (b, 0)),
            pl.BlockSpec((128, d), lambda b, h, tab: (0, 0)),
            pl.BlockSpec((1, 128), lambda b, h, tab: (0, 0)),
            pl.BlockSpec((128, d), lambda b, h, tab: (0, 0)),
            pl.BlockSpec((ht, d), lambda b, h, tab: (h, 0)),
            pl.BlockSpec((1, ht), lambda b, h, tab: (0, h)),
            pl.BlockSpec((ht, d), lambda b, h, tab: (h, 0)),
        ],
        out_specs=pl.BlockSpec((bm, d), lambda b, h, tab: (b, 0)),
        scratch_shapes=[pltpu.VMEM((bm, 128), jnp.float32)],
    )

    out = pl.pallas_call(
        functools.partial(_ffn_body, n_experts=n_experts),
        grid_spec=grid_spec,
        out_shape=jax.ShapeDtypeStruct((n, d), jnp.float32),
        compiler_params=pltpu.CompilerParams(
            dimension_semantics=("arbitrary", "arbitrary")),
    )(jnp.asarray(etab), x, wr_pad, br_pad, b2c, w1cat, b1cat, w2cat)
    return out


# speedup vs baseline: 2.5699x; 1.3755x over previous
import functools

import jax
import jax.numpy as jnp
import numpy as np
from jax import lax
from jax.experimental import pallas as pl
from jax.experimental.pallas import tpu as pltpu

_EPS = 1e-5
_NEG = -1e30


def _ffn_body(etab_ref, x_ref, wr_ref, br_ref, b2c_ref, w1_ref, b1_ref,
              w2_ref, o_ref, coef_scr, x16_scr, *, n_experts):
    h_id = pl.program_id(1)

    @pl.when(h_id == 0)
    def _router():
        x = x_ref[...]
        x16_scr[...] = x.astype(jnp.bfloat16)
        logits = jnp.dot(x, wr_ref[...].T, preferred_element_type=jnp.float32)
        logits = logits + br_ref[...]
        col = lax.broadcasted_iota(jnp.int32, logits.shape, 1)
        lm = jnp.where(col < n_experts, logits, _NEG)
        m1 = jnp.max(lm, axis=1, keepdims=True)
        i1 = jnp.min(jnp.where(lm == m1, col, 127), axis=1, keepdims=True)
        l2 = jnp.where(col == i1, _NEG, lm)
        m2 = jnp.max(l2, axis=1, keepdims=True)
        i2 = jnp.min(jnp.where(l2 == m2, col, 127), axis=1, keepdims=True)
        e = jnp.exp(m2 - m1)
        wa = 1.0 / (1.0 + e)
        wb = 1.0 - wa
        coef = wa * (col == i1) + wb * (col == i2)
        coef_scr[...] = coef
        o_ref[...] = x + jnp.dot(coef, b2c_ref[...],
                                 preferred_element_type=jnp.float32)

    et = etab_ref[h_id]
    coef = coef_scr[...]
    col = lax.broadcasted_iota(jnp.int32, coef.shape, 1)
    csel = jnp.sum(jnp.where(col == et, coef, 0.0), axis=1)
    h = jnp.dot(x16_scr[...], w1_ref[...].T,
                preferred_element_type=jnp.float32)
    h = jnp.maximum(h + b1_ref[...], 0.0) * csel[:, None]
    o_ref[...] += jnp.dot(h.astype(jnp.bfloat16), w2_ref[...],
                          preferred_element_type=jnp.float32)


def kernel(x, router_W, router_b, expert_params):
    n, d = x.shape
    n_experts = len(expert_params)
    inv_c = 1.0 / np.sqrt(1.0 + _EPS)

    bm = min(256, n)
    ht = 512

    w1_parts, b1_parts, w2_parts = [], [], []
    b2_rows = []
    sizes_p = []
    for (W1, b1, g1, be1, W2, b2, g2, be2) in expert_params:
        s = W1.shape[0]
        sp = ((s + ht - 1) // ht) * ht
        sizes_p.append(sp)
        s1 = g1 * inv_c
        w1f = W1 * s1[:, None]
        b1f = b1 * s1 + be1
        s2 = g2 * inv_c
        w2f = (W2 * s2[:, None]).T
        b2f = b2 * s2 + be2
        w1_parts.append(jnp.pad(w1f, ((0, sp - s), (0, 0))))
        b1_parts.append(jnp.pad(b1f, (0, sp - s)))
        w2_parts.append(jnp.pad(w2f, ((0, sp - s), (0, 0))))
        b2_rows.append(b2f)

    w1cat = jnp.concatenate(w1_parts, axis=0).astype(jnp.bfloat16)
    b1cat = jnp.concatenate(b1_parts, axis=0)[None, :]
    w2cat = jnp.concatenate(w2_parts, axis=0).astype(jnp.bfloat16)
    s_tot = w1cat.shape[0]
    nt = s_tot // ht

    etab = np.repeat(np.arange(n_experts, dtype=np.int32),
                     [sp // ht for sp in sizes_p])

    wr_pad = jnp.pad(router_W, ((0, 128 - n_experts), (0, 0)))
    br_pad = jnp.pad(router_b, (0, 128 - n_experts))[None, :]
    b2c = jnp.pad(jnp.stack(b2_rows, axis=0), ((0, 128 - n_experts), (0, 0)))

    nb = n // bm
    grid_spec = pltpu.PrefetchScalarGridSpec(
        num_scalar_prefetch=1,
        grid=(nb, nt),
        in_specs=[
            pl.BlockSpec((bm, d), lambda b, h, tab: (b, 0)),
            pl.BlockSpec((128, d), lambda b, h, tab: (0, 0)),
            pl.BlockSpec((1, 128), lambda b, h, tab: (0, 0)),
            pl.BlockSpec((128, d), lambda b, h, tab: (0, 0)),
            pl.BlockSpec((ht, d), lambda b, h, tab: (h, 0)),
            pl.BlockSpec((1, ht), lambda b, h, tab: (0, h)),
            pl.BlockSpec((ht, d), lambda b, h, tab: (h, 0)),
        ],
        out_specs=pl.BlockSpec((bm, d), lambda b, h, tab: (b, 0)),
        scratch_shapes=[pltpu.VMEM((bm, 128), jnp.float32),
                        pltpu.VMEM((bm, d), jnp.bfloat16)],
    )

    out = pl.pallas_call(
        functools.partial(_ffn_body, n_experts=n_experts),
        grid_spec=grid_spec,
        out_shape=jax.ShapeDtypeStruct((n, d), jnp.float32),
        compiler_params=pltpu.CompilerParams(
            dimension_semantics=("arbitrary", "arbitrary")),
    )(jnp.asarray(etab), x, wr_pad, br_pad, b2c, w1cat, b1cat, w2cat)
    return out


# speedup vs baseline: 3.4485x; 1.3419x over previous
import functools

import jax
import jax.numpy as jnp
import numpy as np
from jax import lax
from jax.experimental import pallas as pl
from jax.experimental.pallas import tpu as pltpu

_EPS = 1e-5
_NEG = -1e30


def _ffn_body(etab_ref, x_ref, wr_ref, br_ref, b2c_ref, w1_ref, b1_ref,
              w2_ref, o_ref, coef_scr, x16_scr, *, n_experts):
    h_id = pl.program_id(1)

    @pl.when(h_id == 0)
    def _router():
        x = x_ref[...]
        x16_scr[...] = x.astype(jnp.bfloat16)
        logits = jnp.dot(x, wr_ref[...].T, preferred_element_type=jnp.float32)
        logits = logits + br_ref[...]
        col = lax.broadcasted_iota(jnp.int32, logits.shape, 1)
        lm = jnp.where(col < n_experts, logits, _NEG)
        m1 = jnp.max(lm, axis=1, keepdims=True)
        i1 = jnp.min(jnp.where(lm == m1, col, 127), axis=1, keepdims=True)
        l2 = jnp.where(col == i1, _NEG, lm)
        m2 = jnp.max(l2, axis=1, keepdims=True)
        i2 = jnp.min(jnp.where(l2 == m2, col, 127), axis=1, keepdims=True)
        e = jnp.exp(m2 - m1)
        wa = 1.0 / (1.0 + e)
        wb = 1.0 - wa
        coef = wa * (col == i1) + wb * (col == i2)
        coef_scr[...] = coef
        o_ref[...] = x + jnp.dot(coef, b2c_ref[...],
                                 preferred_element_type=jnp.float32)

    et = etab_ref[h_id]
    coef = coef_scr[...]
    col = lax.broadcasted_iota(jnp.int32, coef.shape, 1)
    csel = jnp.sum(jnp.where(col == et, coef, 0.0), axis=1)
    h = jnp.dot(x16_scr[...], w1_ref[...].T,
                preferred_element_type=jnp.float32)
    h = jnp.maximum(h + b1_ref[...], 0.0) * csel[:, None]
    o_ref[...] += jnp.dot(h.astype(jnp.bfloat16), w2_ref[...],
                          preferred_element_type=jnp.float32)


def kernel(x, router_W, router_b, expert_params):
    n, d = x.shape
    n_experts = len(expert_params)
    inv_c = 1.0 / np.sqrt(1.0 + _EPS)

    bm = min(1024, n)
    ht = 512

    w1_parts, b1_parts, w2_parts = [], [], []
    b2_rows = []
    sizes_p = []
    for (W1, b1, g1, be1, W2, b2, g2, be2) in expert_params:
        s = W1.shape[0]
        sp = ((s + ht - 1) // ht) * ht
        sizes_p.append(sp)
        s1 = g1 * inv_c
        w1f = (W1 * s1[:, None]).astype(jnp.bfloat16)
        b1f = b1 * s1 + be1
        s2 = g2 * inv_c
        w2f = (W2 * s2[:, None]).T.astype(jnp.bfloat16)
        b2f = b2 * s2 + be2
        w1_parts.append(jnp.pad(w1f, ((0, sp - s), (0, 0))))
        b1_parts.append(jnp.pad(b1f, (0, sp - s)))
        w2_parts.append(jnp.pad(w2f, ((0, sp - s), (0, 0))))
        b2_rows.append(b2f)

    w1cat = jnp.concatenate(w1_parts, axis=0)
    b1cat = jnp.concatenate(b1_parts, axis=0)[None, :]
    w2cat = jnp.concatenate(w2_parts, axis=0)
    s_tot = w1cat.shape[0]
    nt = s_tot // ht

    etab = np.repeat(np.arange(n_experts, dtype=np.int32),
                     [sp // ht for sp in sizes_p])

    wr_pad = jnp.pad(router_W, ((0, 128 - n_experts), (0, 0)))
    br_pad = jnp.pad(router_b, (0, 128 - n_experts))[None, :]
    b2c = jnp.pad(jnp.stack(b2_rows, axis=0), ((0, 128 - n_experts), (0, 0)))

    nb = n // bm
    grid_spec = pltpu.PrefetchScalarGridSpec(
        num_scalar_prefetch=1,
        grid=(nb, nt),
        in_specs=[
            pl.BlockSpec((bm, d), lambda b, h, tab: (b, 0)),
            pl.BlockSpec((128, d), lambda b, h, tab: (0, 0)),
            pl.BlockSpec((1, 128), lambda b, h, tab: (0, 0)),
            pl.BlockSpec((128, d), lambda b, h, tab: (0, 0)),
            pl.BlockSpec((ht, d), lambda b, h, tab: (h, 0)),
            pl.BlockSpec((1, ht), lambda b, h, tab: (0, h)),
            pl.BlockSpec((ht, d), lambda b, h, tab: (h, 0)),
        ],
        out_specs=pl.BlockSpec((bm, d), lambda b, h, tab: (b, 0)),
        scratch_shapes=[pltpu.VMEM((bm, 128), jnp.float32),
                        pltpu.VMEM((bm, d), jnp.bfloat16)],
    )

    out = pl.pallas_call(
        functools.partial(_ffn_body, n_experts=n_experts),
        grid_spec=grid_spec,
        out_shape=jax.ShapeDtypeStruct((n, d), jnp.float32),
        compiler_params=pltpu.CompilerParams(
            dimension_semantics=("arbitrary", "arbitrary")),
    )(jnp.asarray(etab), x, wr_pad, br_pad, b2c, w1cat, b1cat, w2cat)
    return out
